# baseline (speedup 1.0000x reference)
import functools

import jax
import jax.numpy as jnp
from jax import lax
from jax.experimental import pallas as pl
from jax.experimental.pallas import tpu as pltpu
from jax.experimental.pallas import tpu_sc as plsc

B = 4096
V = 100000
D = 128
W2 = 10
NNEG = 20
NPAIR = W2 + NNEG
SLOT = 32
NC, NS, L = 2, 16, 16
NW = NC * NS
BPW = B // NW
CB = 8
NCHUNK = BPW // CB
CP = CB * NPAIR
PROW = CP // 2
DK = D // L


def _sc_dots_kernel(xi_hbm, yn_hbm, ein_hbm, eout_hbm, out_hbm,
                    xidx_all, ynidx_all,
                    xrows0, ynrows0, dbuf0, xrows1, ynrows1, dbuf1,
                    part, sem_r0, sem_x0, sem_o0, sem_r1, sem_x1, sem_o1):
    wid = lax.axis_index("s") * NC + lax.axis_index("c")
    lane = lax.iota(jnp.int32, L)
    zeros = jnp.zeros((L,), jnp.float32)
    part[pl.ds(NPAIR * L, L)] = zeros
    part[pl.ds((NPAIR + 1) * L, L)] = zeros

    pltpu.sync_copy(xi_hbm.at[pl.ds(pl.multiple_of(wid * BPW, BPW), BPW)],
                    xidx_all)
    pltpu.sync_copy(
        yn_hbm.at[pl.ds(pl.multiple_of(wid * BPW * NPAIR, BPW * NPAIR),
                        BPW * NPAIR)], ynidx_all)

    bufs = ((xrows0, ynrows0, dbuf0, sem_r0, sem_x0, sem_o0),
            (xrows1, ynrows1, dbuf1, sem_r1, sem_x1, sem_o1))

    def fire(g, buf):
        xrows, ynrows, dbuf, sem_r, sem_x, sem_o = buf
        o = pl.multiple_of(g * CP, CP)
        pltpu.async_copy(eout_hbm.at[ynidx_all.at[pl.ds(o, PROW)]],
                         ynrows.at[pl.ds(0, PROW)], sem_r)
        o2 = pl.multiple_of(g * CP + PROW, PROW)
        pltpu.async_copy(eout_hbm.at[ynidx_all.at[pl.ds(o2, PROW)]],
                         ynrows.at[pl.ds(PROW, PROW)], sem_r)
        ox = pl.multiple_of(g * CB, CB)
        pltpu.async_copy(ein_hbm.at[xidx_all.at[pl.ds(ox, CB)]],
                         xrows, sem_x)

    def drain(buf):
        xrows, ynrows, dbuf, sem_r, sem_x, sem_o = buf
        pltpu.make_async_copy(eout_hbm.at[pl.ds(0, CP)],
                              ynrows, sem_r).wait()
        pltpu.make_async_copy(ein_hbm.at[pl.ds(0, CB)], xrows, sem_x).wait()

    def drain_out(buf):
        xrows, ynrows, dbuf, sem_r, sem_x, sem_o = buf
        pltpu.make_async_copy(dbuf, out_hbm.at[pl.ds(0, CB * SLOT)],
                              sem_o).wait()

    def compute(g, buf):
        xrows, ynrows, dbuf, sem_r, sem_x, sem_o = buf
        base = wid * BPW + g * CB

        def b_body(bi, bcarry):
            xk = [xrows[bi, pl.ds(L * k, L)] for k in range(DK)]

            def load8(j):
                p = bi * NPAIR + j
                return [ynrows[p, pl.ds(L * k, L)] for k in range(DK)]

            rows = load8(0)
            for j in range(NPAIR):
                cur = rows
                if j + 1 < NPAIR:
                    rows = load8(j + 1)
                t = [cur[k] * xk[k] for k in range(DK)]
                part[pl.ds(j * L, L)] = (
                    ((t[0] + t[1]) + (t[2] + t[3]))
                    + ((t[4] + t[5]) + (t[6] + t[7])))
            lanL = lane * L
            g0 = [plsc.load_gather(part, [lanL + l]) for l in range(L)]
            g1 = [plsc.load_gather(part, [lanL + (L * L + l)])
                  for l in range(L)]
            while len(g0) > 1:
                g0 = [g0[i] + g0[i + 1] for i in range(0, len(g0), 2)]
                g1 = [g1[i] + g1[i + 1] for i in range(0, len(g1), 2)]
            out0 = g0[0]
            out1 = g1[0]
            yv = plsc.load_gather(ynidx_all, [g * CP + bi * NPAIR + lane])
            msk = (yv == 0) & (lane < W2)
            out0 = jnp.where(msk, jnp.float32(jnp.inf), out0)
            off = pl.multiple_of(bi * SLOT, SLOT)
            dbuf[pl.ds(off, L)] = out0
            off2 = pl.multiple_of(bi * SLOT + L, L)
            dbuf[pl.ds(off2, L)] = out1
            return bcarry

        lax.fori_loop(0, CB, b_body, 0)
        pltpu.async_copy(
            dbuf, out_hbm.at[pl.ds(pl.multiple_of(base * SLOT, CB * SLOT),
                                   CB * SLOT)], sem_o)

    fire(0, bufs[0])

    def h_body(h, carry):
        g = h * 2
        fire(g + 1, bufs[1])
        drain(bufs[0])

        @pl.when(g >= 2)
        def _():
            drain_out(bufs[0])

        compute(g, bufs[0])

        @pl.when(g + 2 < NCHUNK)
        def _():
            fire(g + 2, bufs[0])

        drain(bufs[1])

        @pl.when(g >= 2)
        def _():
            drain_out(bufs[1])

        compute(g + 1, bufs[1])
        return carry

    lax.fori_loop(0, NCHUNK // 2, h_body, 0)
    drain_out(bufs[0])
    drain_out(bufs[1])


_sc_dots = functools.partial(
    pl.kernel,
    out_type=jax.ShapeDtypeStruct((B * SLOT,), jnp.float32),
    mesh=plsc.VectorSubcoreMesh(core_axis_name="c", subcore_axis_name="s",
                                num_cores=NC, num_subcores=NS),
    scratch_types=(
        [pltpu.VMEM((BPW,), jnp.int32),
         pltpu.VMEM((BPW * NPAIR,), jnp.int32)]
        + [pltpu.VMEM((CB, D), jnp.float32),
           pltpu.VMEM((CP, D), jnp.float32),
           pltpu.VMEM((CB * SLOT,), jnp.float32),
           ] * 2
        + [pltpu.VMEM((SLOT * L,), jnp.float32),
           pltpu.SemaphoreType.DMA, pltpu.SemaphoreType.DMA,
           pltpu.SemaphoreType.DMA, pltpu.SemaphoreType.DMA,
           pltpu.SemaphoreType.DMA, pltpu.SemaphoreType.DMA]),
    compiler_params=pltpu.CompilerParams(needs_layout_passes=False),
)(_sc_dots_kernel)


def _tc_loss_kernel(d_ref, o_ref):
    x = d_ref[...]
    slot = lax.broadcasted_iota(jnp.int32, (B * SLOT,), 0) % SLOT
    s = jnp.where(slot < W2, x, -x)
    v = jnp.log(jax.nn.sigmoid(s))
    v = jnp.where(slot < NPAIR, v, 0.0)
    o_ref[0, 0] = -jnp.sum(v) / B


_tc_loss = pl.pallas_call(
    _tc_loss_kernel,
    out_shape=jax.ShapeDtypeStruct((1, 1), jnp.float32),
    out_specs=pl.BlockSpec(memory_space=pltpu.SMEM),
)


def kernel(batch_X, batch_Y, batch_N, emb_in, emb_out):
    bx = batch_X.astype(jnp.int32)
    yn = jnp.concatenate([batch_Y, batch_N], axis=1).astype(jnp.int32)
    yn = yn.reshape(B * NPAIR)
    dots = _sc_dots(bx, yn, emb_in, emb_out)
    return _tc_loss(dots)[0, 0]


# speedup vs baseline: 1.0374x; 1.0374x over previous
import functools

import jax
import jax.numpy as jnp
from jax import lax
from jax.experimental import pallas as pl
from jax.experimental.pallas import tpu as pltpu
from jax.experimental.pallas import tpu_sc as plsc

B = 4096
V = 100000
D = 128
W2 = 10
NNEG = 20
NPAIR = W2 + NNEG
SLOT = 32
NC, NS, L = 2, 16, 16
NW = NC * NS
BPW = B // NW
CB = 8
NCHUNK = BPW // CB
CP = CB * NPAIR
PROW = CP // 2
DK = D // L


def _sc_dots_kernel(xi_hbm, yn_hbm, ein_hbm, eout_hbm, out_hbm,
                    xidx_all, ynidx_all,
                    xrows0, ynrows0, dbuf0, xrows1, ynrows1, dbuf1,
                    part, sem_r0, sem_x0, sem_o0, sem_r1, sem_x1, sem_o1):
    wid = lax.axis_index("s") * NC + lax.axis_index("c")
    lane = lax.iota(jnp.int32, L)
    zeros = jnp.zeros((L,), jnp.float32)
    part[pl.ds(NPAIR * L, L)] = zeros
    part[pl.ds((NPAIR + 1) * L, L)] = zeros

    pltpu.sync_copy(xi_hbm.at[pl.ds(pl.multiple_of(wid * BPW, BPW), BPW)],
                    xidx_all)
    pltpu.sync_copy(
        yn_hbm.at[pl.ds(pl.multiple_of(wid * BPW * NPAIR, BPW * NPAIR),
                        BPW * NPAIR)], ynidx_all)

    bufs = ((xrows0, ynrows0, dbuf0, sem_r0, sem_x0, sem_o0),
            (xrows1, ynrows1, dbuf1, sem_r1, sem_x1, sem_o1))

    def fire(g, buf):
        xrows, ynrows, dbuf, sem_r, sem_x, sem_o = buf
        o = pl.multiple_of(g * CP, CP)
        pltpu.async_copy(eout_hbm.at[ynidx_all.at[pl.ds(o, PROW)]],
                         ynrows.at[pl.ds(0, PROW)], sem_r)
        o2 = pl.multiple_of(g * CP + PROW, PROW)
        pltpu.async_copy(eout_hbm.at[ynidx_all.at[pl.ds(o2, PROW)]],
                         ynrows.at[pl.ds(PROW, PROW)], sem_r)
        ox = pl.multiple_of(g * CB, CB)
        pltpu.async_copy(ein_hbm.at[xidx_all.at[pl.ds(ox, CB)]],
                         xrows, sem_x)

    def drain(buf):
        xrows, ynrows, dbuf, sem_r, sem_x, sem_o = buf
        pltpu.make_async_copy(eout_hbm.at[pl.ds(0, CP)],
                              ynrows, sem_r).wait()
        pltpu.make_async_copy(ein_hbm.at[pl.ds(0, CB)], xrows, sem_x).wait()

    def drain_out(buf):
        xrows, ynrows, dbuf, sem_r, sem_x, sem_o = buf
        pltpu.make_async_copy(dbuf, out_hbm.at[pl.ds(0, CB * SLOT)],
                              sem_o).wait()

    def compute(g, buf):
        xrows, ynrows, dbuf, sem_r, sem_x, sem_o = buf
        base = wid * BPW + g * CB

        def b_body(bi, bcarry):
            xk = [xrows[bi, pl.ds(L * k, L)] for k in range(DK)]

            def load8(j):
                p = bi * NPAIR + j
                return [ynrows[p, pl.ds(L * k, L)] for k in range(DK)]

            rows = load8(0)
            for j in range(NPAIR):
                cur = rows
                if j + 1 < NPAIR:
                    rows = load8(j + 1)
                t = [cur[k] * xk[k] for k in range(DK)]
                part[pl.ds(j * L, L)] = (
                    ((t[0] + t[1]) + (t[2] + t[3]))
                    + ((t[4] + t[5]) + (t[6] + t[7])))
            lanL = lane * L
            g0 = [plsc.load_gather(part, [lanL + l]) for l in range(L)]
            g1 = [plsc.load_gather(part, [lanL + (L * L + l)])
                  for l in range(L)]
            while len(g0) > 1:
                g0 = [g0[i] + g0[i + 1] for i in range(0, len(g0), 2)]
                g1 = [g1[i] + g1[i + 1] for i in range(0, len(g1), 2)]
            out0 = g0[0]
            out1 = g1[0]
            yv = plsc.load_gather(ynidx_all, [g * CP + bi * NPAIR + lane])
            msk = (yv == 0) & (lane < W2)
            out0 = jnp.where(msk, jnp.float32(jnp.inf), out0)
            off = pl.multiple_of(bi * SLOT, SLOT)
            dbuf[pl.ds(off, L)] = out0
            off2 = pl.multiple_of(bi * SLOT + L, L)
            dbuf[pl.ds(off2, L)] = out1
            return bcarry

        lax.fori_loop(0, CB, b_body, 0)
        pltpu.async_copy(
            dbuf, out_hbm.at[pl.ds(pl.multiple_of(base * SLOT, CB * SLOT),
                                   CB * SLOT)], sem_o)

    fire(0, bufs[0])

    def h_body(h, carry):
        g = h * 2
        fire(g + 1, bufs[1])
        drain(bufs[0])

        @pl.when(g >= 2)
        def _():
            drain_out(bufs[0])

        compute(g, bufs[0])

        @pl.when(g + 2 < NCHUNK)
        def _():
            fire(g + 2, bufs[0])

        drain(bufs[1])

        @pl.when(g >= 2)
        def _():
            drain_out(bufs[1])

        compute(g + 1, bufs[1])
        return carry

    lax.fori_loop(0, NCHUNK // 2, h_body, 0)
    drain_out(bufs[0])
    drain_out(bufs[1])


_sc_dots = functools.partial(
    pl.kernel,
    out_type=jax.ShapeDtypeStruct((B * SLOT,), jnp.float32),
    mesh=plsc.VectorSubcoreMesh(core_axis_name="c", subcore_axis_name="s",
                                num_cores=NC, num_subcores=NS),
    scratch_types=(
        [pltpu.VMEM((BPW,), jnp.int32),
         pltpu.VMEM((BPW * NPAIR,), jnp.int32)]
        + [pltpu.VMEM((CB, D), jnp.float32),
           pltpu.VMEM((CP, D), jnp.float32),
           pltpu.VMEM((CB * SLOT,), jnp.float32),
           ] * 2
        + [pltpu.VMEM((SLOT * L,), jnp.float32),
           pltpu.SemaphoreType.DMA, pltpu.SemaphoreType.DMA,
           pltpu.SemaphoreType.DMA, pltpu.SemaphoreType.DMA,
           pltpu.SemaphoreType.DMA, pltpu.SemaphoreType.DMA]),
    compiler_params=pltpu.CompilerParams(needs_layout_passes=False),
)(_sc_dots_kernel)


ROWS = B * SLOT // 128


def _tc_loss_kernel(d_ref, o_ref):
    x = d_ref[...]
    slot = lax.broadcasted_iota(jnp.int32, (ROWS, 128), 1) % SLOT
    s = jnp.where(slot < W2, x, -x)
    v = jnp.log(jax.nn.sigmoid(s))
    v = jnp.where(slot < NPAIR, v, 0.0)
    o_ref[0, 0] = -jnp.sum(v) / B


_tc_loss = pl.pallas_call(
    _tc_loss_kernel,
    out_shape=jax.ShapeDtypeStruct((1, 1), jnp.float32),
    out_specs=pl.BlockSpec(memory_space=pltpu.SMEM),
)


def kernel(batch_X, batch_Y, batch_N, emb_in, emb_out):
    bx = batch_X.astype(jnp.int32)
    yn = jnp.concatenate([batch_Y, batch_N], axis=1).astype(jnp.int32)
    yn = yn.reshape(B * NPAIR)
    dots = _sc_dots(bx, yn, emb_in, emb_out)
    return _tc_loss(dots.reshape(ROWS, 128))[0, 0]
